# Initial kernel scaffold; baseline (speedup 1.0000x reference)
import jax
import jax.numpy as jnp
from jax.experimental import pallas as pl


def kernel(q_lora, hidden_states, positions, Wq_b, Wk, k_gamma, k_beta):
    raise NotImplementedError("write your pallas kernel here")


# speedup vs baseline: 11.2695x; 11.2695x over previous
import functools

import numpy as np
import jax
import jax.numpy as jnp
from jax.experimental import pallas as pl

T = 8192
HIDDEN = 2048
NHEADS = 16
HEAD_DIM = 128
ROPE_DIM = 64
QLORA = 1536
ROPE_THETA = 10000.0

BM = 256


def _hadamard128():
    h = np.array([[1.0]], dtype=np.float64)
    while h.shape[0] < HEAD_DIM:
        h = np.block([[h, h], [h, -h]])
    return (h * (HEAD_DIM ** -0.5)).astype(np.float32)


_H128 = _hadamard128()

_PERM_HALF = np.concatenate(
    [np.arange(0, ROPE_DIM, 2), np.arange(1, ROPE_DIM, 2), np.arange(ROPE_DIM, HEAD_DIM)]
)
_INV_FREQ = (
    1.0 / (ROPE_THETA ** (np.arange(0, ROPE_DIM, 2).astype(np.float32) / ROPE_DIM))
).reshape(1, ROPE_DIM // 2)


def _indexer_kernel(ql_ref, hid_ref, pos_ref, wq_ref, wk_ref, gam_ref, bet_ref,
                    ifreq_ref, hmat_ref, q_out_ref, k_out_ref):
    half = ROPE_DIM // 2
    pos = pos_ref[...].astype(jnp.float32)
    freqs = pos * ifreq_ref[...]
    cos = jnp.cos(freqs)
    sin = jnp.sin(freqs)
    hmat = hmat_ref[...]

    def rope_then_h(x):
        x1 = x[:, :half]
        x2 = x[:, half:ROPE_DIM]
        rot = jnp.concatenate(
            [x1 * cos - x2 * sin, x2 * cos + x1 * sin, x[:, ROPE_DIM:]], axis=1)
        return jnp.dot(rot, hmat, preferred_element_type=jnp.float32)

    k = jnp.dot(hid_ref[...], wk_ref[...], preferred_element_type=jnp.float32)
    mu = jnp.mean(k, axis=1, keepdims=True)
    var = jnp.mean((k - mu) ** 2, axis=1, keepdims=True)
    k = (k - mu) * jax.lax.rsqrt(var + 1e-5) * gam_ref[...] + bet_ref[...]
    k_out_ref[...] = rope_then_h(k)

    q = jnp.dot(ql_ref[...], wq_ref[...], preferred_element_type=jnp.float32)
    heads = []
    for h in range(NHEADS):
        heads.append(rope_then_h(q[:, h * HEAD_DIM:(h + 1) * HEAD_DIM]))
    q_out_ref[...] = jnp.concatenate(heads, axis=1)


@jax.jit
def kernel(q_lora, hidden_states, positions, Wq_b, Wk, k_gamma, k_beta):
    nt = q_lora.shape[0]
    qperm = (np.arange(NHEADS)[:, None] * HEAD_DIM + _PERM_HALF[None, :]).reshape(-1)
    wq = Wq_b[:, qperm]
    wk = Wk[:, _PERM_HALF]
    gam = k_gamma[_PERM_HALF].reshape(1, HEAD_DIM)
    bet = k_beta[_PERM_HALF].reshape(1, HEAD_DIM)
    pos2d = positions.reshape(nt, 1)

    grid = (nt // BM,)
    q2d, key = pl.pallas_call(
        _indexer_kernel,
        grid=grid,
        in_specs=[
            pl.BlockSpec((BM, QLORA), lambda i: (i, 0)),
            pl.BlockSpec((BM, HIDDEN), lambda i: (i, 0)),
            pl.BlockSpec((BM, 1), lambda i: (i, 0)),
            pl.BlockSpec((QLORA, NHEADS * HEAD_DIM), lambda i: (0, 0)),
            pl.BlockSpec((HIDDEN, HEAD_DIM), lambda i: (0, 0)),
            pl.BlockSpec((1, HEAD_DIM), lambda i: (0, 0)),
            pl.BlockSpec((1, HEAD_DIM), lambda i: (0, 0)),
            pl.BlockSpec((1, ROPE_DIM // 2), lambda i: (0, 0)),
            pl.BlockSpec((HEAD_DIM, HEAD_DIM), lambda i: (0, 0)),
        ],
        out_specs=[
            pl.BlockSpec((BM, NHEADS * HEAD_DIM), lambda i: (i, 0)),
            pl.BlockSpec((BM, HEAD_DIM), lambda i: (i, 0)),
        ],
        out_shape=[
            jax.ShapeDtypeStruct((nt, NHEADS * HEAD_DIM), jnp.float32),
            jax.ShapeDtypeStruct((nt, HEAD_DIM), jnp.float32),
        ],
    )(q_lora, hidden_states, pos2d, wq, wk, gam, bet,
      jnp.asarray(_INV_FREQ), jnp.asarray(_H128))
    return q2d.reshape(nt, NHEADS, HEAD_DIM), key


# speedup vs baseline: 12.7013x; 1.1270x over previous
import functools

import numpy as np
import jax
import jax.numpy as jnp
from jax.experimental import pallas as pl

T = 8192
HIDDEN = 2048
NHEADS = 16
HEAD_DIM = 128
ROPE_DIM = 64
QLORA = 1536
ROPE_THETA = 10000.0

BM = 256


def _hadamard128():
    h = np.array([[1.0]], dtype=np.float64)
    while h.shape[0] < HEAD_DIM:
        h = np.block([[h, h], [h, -h]])
    return h.astype(np.float32)


_H128 = _hadamard128()

_PERM_HALF = np.concatenate(
    [np.arange(0, ROPE_DIM, 2), np.arange(1, ROPE_DIM, 2), np.arange(ROPE_DIM, HEAD_DIM)]
)
_INV_FREQ = (
    1.0 / (ROPE_THETA ** (np.arange(0, ROPE_DIM, 2).astype(np.float32) / ROPE_DIM))
).reshape(1, ROPE_DIM // 2)


def _indexer_kernel(ql_ref, hid_ref, pos_ref, wq_ref, wk_ref, gam_ref, bet_ref,
                    ifreq_ref, hmat_ref, q_out_ref, k_out_ref):
    half = ROPE_DIM // 2
    pos = pos_ref[...].astype(jnp.float32)
    freqs = pos * ifreq_ref[...]
    cos = jnp.cos(freqs)
    sin = jnp.sin(freqs)
    hmat = hmat_ref[...]

    def rope_then_h(x):
        x1 = x[:, :half]
        x2 = x[:, half:ROPE_DIM]
        rot = jnp.concatenate(
            [x1 * cos - x2 * sin, x2 * cos + x1 * sin, x[:, ROPE_DIM:]], axis=1)
        return jnp.dot(rot.astype(jnp.bfloat16), hmat,
                       preferred_element_type=jnp.float32) * (HEAD_DIM ** -0.5)

    k = jnp.dot(hid_ref[...].astype(jnp.bfloat16), wk_ref[...],
                preferred_element_type=jnp.float32)
    mu = jnp.mean(k, axis=1, keepdims=True)
    var = jnp.mean((k - mu) ** 2, axis=1, keepdims=True)
    k = (k - mu) * jax.lax.rsqrt(var + 1e-5) * gam_ref[...] + bet_ref[...]
    k_out_ref[...] = rope_then_h(k)

    q = jnp.dot(ql_ref[...].astype(jnp.bfloat16), wq_ref[...],
                preferred_element_type=jnp.float32)
    heads = []
    for h in range(NHEADS):
        heads.append(rope_then_h(q[:, h * HEAD_DIM:(h + 1) * HEAD_DIM]))
    q_out_ref[...] = jnp.concatenate(heads, axis=1)


@jax.jit
def kernel(q_lora, hidden_states, positions, Wq_b, Wk, k_gamma, k_beta):
    nt = q_lora.shape[0]
    qperm = (np.arange(NHEADS)[:, None] * HEAD_DIM + _PERM_HALF[None, :]).reshape(-1)
    wq = Wq_b[:, qperm].astype(jnp.bfloat16)
    wk = Wk[:, _PERM_HALF].astype(jnp.bfloat16)
    gam = k_gamma[_PERM_HALF].reshape(1, HEAD_DIM)
    bet = k_beta[_PERM_HALF].reshape(1, HEAD_DIM)
    pos2d = positions.reshape(nt, 1)

    grid = (nt // BM,)
    q2d, key = pl.pallas_call(
        _indexer_kernel,
        grid=grid,
        in_specs=[
            pl.BlockSpec((BM, QLORA), lambda i: (i, 0)),
            pl.BlockSpec((BM, HIDDEN), lambda i: (i, 0)),
            pl.BlockSpec((BM, 1), lambda i: (i, 0)),
            pl.BlockSpec((QLORA, NHEADS * HEAD_DIM), lambda i: (0, 0)),
            pl.BlockSpec((HIDDEN, HEAD_DIM), lambda i: (0, 0)),
            pl.BlockSpec((1, HEAD_DIM), lambda i: (0, 0)),
            pl.BlockSpec((1, HEAD_DIM), lambda i: (0, 0)),
            pl.BlockSpec((1, ROPE_DIM // 2), lambda i: (0, 0)),
            pl.BlockSpec((HEAD_DIM, HEAD_DIM), lambda i: (0, 0)),
        ],
        out_specs=[
            pl.BlockSpec((BM, NHEADS * HEAD_DIM), lambda i: (i, 0)),
            pl.BlockSpec((BM, HEAD_DIM), lambda i: (i, 0)),
        ],
        out_shape=[
            jax.ShapeDtypeStruct((nt, NHEADS * HEAD_DIM), jnp.float32),
            jax.ShapeDtypeStruct((nt, HEAD_DIM), jnp.float32),
        ],
    )(q_lora, hidden_states, pos2d, wq, wk, gam, bet,
      jnp.asarray(_INV_FREQ), jnp.asarray(_H128, dtype=jnp.bfloat16))
    return q2d.reshape(nt, NHEADS, HEAD_DIM), key
